# Initial kernel scaffold; baseline (speedup 1.0000x reference)
import jax
import jax.numpy as jnp
from jax.experimental import pallas as pl


def kernel(input, table, W1, b1, W2, b2, W3, b3):
    raise NotImplementedError("write your pallas kernel here")


# speedup vs baseline: 6.5165x; 6.5165x over previous
import functools

import jax
import jax.numpy as jnp
from jax import lax
from jax.experimental import pallas as pl
from jax.experimental.pallas import tpu as pltpu
from jax.experimental.pallas import tpu_sc as plsc

VOCAB = 100000
HIDDEN = 128
LABELS = 100
BATCH = 4096
SEQ = 200

NC = 2
NS = 16
NW = NC * NS
ROWS_PER_W = BATCH // NW
HALF = SEQ // 2
NCHUNK = HIDDEN // 16


def _pooled_sum_sc(table, idx3):
    mesh = plsc.VectorSubcoreMesh(
        core_axis_name="c", subcore_axis_name="s", num_cores=NC, num_subcores=NS
    )

    @functools.partial(
        pl.kernel,
        out_type=jax.ShapeDtypeStruct((BATCH, HIDDEN), jnp.float32),
        mesh=mesh,
        scratch_types=[
            pltpu.VMEM((2, HALF), jnp.int32),
            pltpu.VMEM((2, HALF, HIDDEN), jnp.float32),
            pltpu.VMEM((ROWS_PER_W, HIDDEN), jnp.float32),
            pltpu.SemaphoreType.DMA,
        ],
    )
    def k(table_hbm, idx_hbm, out_hbm, idx_v, rows_v, out_v, sem):
        wid = lax.axis_index("s") * NC + lax.axis_index("c")
        base = wid * ROWS_PER_W

        def row_body(i, carry):
            pltpu.sync_copy(idx_hbm.at[base + i], idx_v)
            cp0 = pltpu.async_copy(table_hbm.at[idx_v.at[0]], rows_v.at[0], sem)
            cp1 = pltpu.async_copy(table_hbm.at[idx_v.at[1]], rows_v.at[1], sem)
            cp0.wait()
            cp1.wait()

            def acc_body(j, acc):
                return tuple(
                    acc[c]
                    + rows_v[0, j, pl.ds(16 * c, 16)]
                    + rows_v[1, j, pl.ds(16 * c, 16)]
                    for c in range(NCHUNK)
                )

            acc0 = tuple(jnp.zeros((16,), jnp.float32) for _ in range(NCHUNK))
            acc = lax.fori_loop(0, HALF, acc_body, acc0)
            for c in range(NCHUNK):
                out_v[i, pl.ds(16 * c, 16)] = acc[c]
            return carry

        lax.fori_loop(0, ROWS_PER_W, row_body, 0)
        pltpu.sync_copy(out_v, out_hbm.at[pl.ds(base, ROWS_PER_W)])

    return k(table, idx3)


def _mlp_tc(pooled, inp, t0, W1t, b1, W2t, b2, W3t, b3):
    BB = 512

    def mlp_kernel(p_ref, idx_ref, t0_ref, w1_ref, b1_ref, w2_ref, b2_ref,
                   w3_ref, b3_ref, o_ref):
        cnt = jnp.sum((idx_ref[...] == 0).astype(jnp.float32), axis=1,
                      keepdims=True)
        bow = (p_ref[...] - cnt * t0_ref[...]) * (1.0 / SEQ)
        h = jnp.maximum(
            jnp.dot(bow, w1_ref[...], preferred_element_type=jnp.float32)
            + b1_ref[...], 0.0)
        h = jnp.maximum(
            jnp.dot(h, w2_ref[...], preferred_element_type=jnp.float32)
            + b2_ref[...], 0.0)
        o_ref[...] = (
            jnp.dot(h, w3_ref[...], preferred_element_type=jnp.float32)
            + b3_ref[...])

    return pl.pallas_call(
        mlp_kernel,
        grid=(BATCH // BB,),
        in_specs=[
            pl.BlockSpec((BB, HIDDEN), lambda i: (i, 0)),
            pl.BlockSpec((BB, SEQ), lambda i: (i, 0)),
            pl.BlockSpec((1, HIDDEN), lambda i: (0, 0)),
            pl.BlockSpec((HIDDEN, HIDDEN), lambda i: (0, 0)),
            pl.BlockSpec((1, HIDDEN), lambda i: (0, 0)),
            pl.BlockSpec((HIDDEN, HIDDEN), lambda i: (0, 0)),
            pl.BlockSpec((1, HIDDEN), lambda i: (0, 0)),
            pl.BlockSpec((HIDDEN, LABELS), lambda i: (0, 0)),
            pl.BlockSpec((1, LABELS), lambda i: (0, 0)),
        ],
        out_specs=pl.BlockSpec((BB, LABELS), lambda i: (i, 0)),
        out_shape=jax.ShapeDtypeStruct((BATCH, LABELS), jnp.float32),
    )(pooled, inp, t0, W1t, b1, W2t, b2, W3t, b3)


def kernel(input, table, W1, b1, W2, b2, W3, b3):
    inp = input.astype(jnp.int32)
    idx3 = inp.reshape(BATCH, 2, HALF)
    pooled = _pooled_sum_sc(table, idx3)
    return _mlp_tc(
        pooled, inp, table[0:1],
        W1.T, b1.reshape(1, HIDDEN),
        W2.T, b2.reshape(1, HIDDEN),
        W3.T, b3.reshape(1, LABELS),
    )


# speedup vs baseline: 13.0191x; 1.9979x over previous
import functools

import jax
import jax.numpy as jnp
from jax import lax
from jax.experimental import pallas as pl
from jax.experimental.pallas import tpu as pltpu
from jax.experimental.pallas import tpu_sc as plsc

VOCAB = 100000
HIDDEN = 128
LABELS = 100
BATCH = 4096
SEQ = 200

NC = 2
NS = 16
NW = NC * NS
ROWS_PER_W = BATCH // NW
HALF = SEQ // 2
NCHUNK = HIDDEN // 16


def _pooled_sum_sc(table, idx3):
    mesh = plsc.VectorSubcoreMesh(
        core_axis_name="c", subcore_axis_name="s", num_cores=NC, num_subcores=NS
    )

    @functools.partial(
        pl.kernel,
        out_type=jax.ShapeDtypeStruct((BATCH, HIDDEN), jnp.float32),
        mesh=mesh,
        scratch_types=[
            pltpu.VMEM((ROWS_PER_W, 2, HALF), jnp.int32),
            pltpu.VMEM((2, 2, HALF, HIDDEN), jnp.float32),
            pltpu.VMEM((ROWS_PER_W, HIDDEN), jnp.float32),
            pltpu.SemaphoreType.DMA,
            pltpu.SemaphoreType.DMA,
        ],
    )
    def k(table_hbm, idx_hbm, out_hbm, idx_v, rows_v, out_v, semA, semB):
        wid = lax.axis_index("s") * NC + lax.axis_index("c")
        base = wid * ROWS_PER_W

        pltpu.sync_copy(idx_hbm.at[wid], idx_v)

        def start(i, p, sem):
            for h in range(2):
                pltpu.async_copy(
                    table_hbm.at[idx_v.at[i, h]], rows_v.at[p, h], sem)

        def wait(i, p, sem):
            for h in range(2):
                pltpu.make_async_copy(
                    table_hbm.at[idx_v.at[i, h]], rows_v.at[p, h], sem).wait()

        def acc_row(i, p):
            def acc_body(j, acc):
                return tuple(
                    acc[c]
                    + rows_v[p, 0, j, pl.ds(16 * c, 16)]
                    + rows_v[p, 1, j, pl.ds(16 * c, 16)]
                    for c in range(NCHUNK)
                )

            acc0 = tuple(jnp.zeros((16,), jnp.float32) for _ in range(NCHUNK))
            acc = lax.fori_loop(0, HALF, acc_body, acc0)
            for c in range(NCHUNK):
                out_v[i, pl.ds(16 * c, 16)] = acc[c]

        start(0, 0, semA)

        def pair_body(kk, carry):
            i0 = 2 * kk
            start(i0 + 1, 1, semB)
            wait(i0, 0, semA)
            acc_row(i0, 0)

            @pl.when(i0 + 2 < ROWS_PER_W)
            def _():
                start(i0 + 2, 0, semA)

            wait(i0 + 1, 1, semB)
            acc_row(i0 + 1, 1)
            return carry

        lax.fori_loop(0, ROWS_PER_W // 2, pair_body, 0)
        pltpu.sync_copy(out_v, out_hbm.at[pl.ds(base, ROWS_PER_W)])

    return k(table, idx3)


def _mlp_tc(pooled, inp, t0, W1t, b1, W2t, b2, W3t, b3):
    BB = 512

    def mlp_kernel(p_ref, idx_ref, t0_ref, w1_ref, b1_ref, w2_ref, b2_ref,
                   w3_ref, b3_ref, o_ref):
        cnt = jnp.sum((idx_ref[...] == 0).astype(jnp.float32), axis=1,
                      keepdims=True)
        bow = (p_ref[...] - cnt * t0_ref[...]) * (1.0 / SEQ)
        h = jnp.maximum(
            jnp.dot(bow, w1_ref[...], preferred_element_type=jnp.float32)
            + b1_ref[...], 0.0)
        h = jnp.maximum(
            jnp.dot(h, w2_ref[...], preferred_element_type=jnp.float32)
            + b2_ref[...], 0.0)
        o_ref[...] = (
            jnp.dot(h, w3_ref[...], preferred_element_type=jnp.float32)
            + b3_ref[...])

    return pl.pallas_call(
        mlp_kernel,
        grid=(BATCH // BB,),
        in_specs=[
            pl.BlockSpec((BB, HIDDEN), lambda i: (i, 0)),
            pl.BlockSpec((BB, SEQ), lambda i: (i, 0)),
            pl.BlockSpec((1, HIDDEN), lambda i: (0, 0)),
            pl.BlockSpec((HIDDEN, HIDDEN), lambda i: (0, 0)),
            pl.BlockSpec((1, HIDDEN), lambda i: (0, 0)),
            pl.BlockSpec((HIDDEN, HIDDEN), lambda i: (0, 0)),
            pl.BlockSpec((1, HIDDEN), lambda i: (0, 0)),
            pl.BlockSpec((HIDDEN, LABELS), lambda i: (0, 0)),
            pl.BlockSpec((1, LABELS), lambda i: (0, 0)),
        ],
        out_specs=pl.BlockSpec((BB, LABELS), lambda i: (i, 0)),
        out_shape=jax.ShapeDtypeStruct((BATCH, LABELS), jnp.float32),
    )(pooled, inp, t0, W1t, b1, W2t, b2, W3t, b3)


def kernel(input, table, W1, b1, W2, b2, W3, b3):
    inp = input.astype(jnp.int32)
    idx3 = inp.reshape(NW, ROWS_PER_W, 2, HALF)
    pooled = _pooled_sum_sc(table, idx3)
    return _mlp_tc(
        pooled, inp, table[0:1],
        W1.T, b1.reshape(1, HIDDEN),
        W2.T, b2.reshape(1, HIDDEN),
        W3.T, b3.reshape(1, LABELS),
    )
